# baseline (speedup 1.0000x reference)
import functools

import jax
import jax.numpy as jnp
from jax import lax
from jax.experimental import pallas as pl
from jax.experimental.pallas import tpu as pltpu
from jax.experimental.pallas import tpu_sc as plsc

DIM_ = 2048
ROWS_ = 4 * 4096
BLOCK_ROWS_ = 1024

NW_ = 32
RPW_ = ROWS_ // NW_
CH_ = 16
NCH_ = RPW_ // CH_
NVREG_ = DIM_ // 16


def _sc_body(x_hbm, emb_hbm, idx_hbm, out_hbm, idxv, tagblk, buf, sem):
    wid = lax.axis_index("s") * 2 + lax.axis_index("c")
    base = wid * RPW_
    pltpu.sync_copy(idx_hbm, idxv)
    pltpu.async_copy(emb_hbm.at[idxv], tagblk, sem).wait()

    def chunk_body(i, _):
        off = base + i * CH_
        pltpu.sync_copy(x_hbm.at[pl.ds(off, CH_)], buf)

        def vec_body(v, _):
            s = v * 16
            t = tagblk[0, pl.ds(s, 16)]
            for r in range(CH_):
                buf[r, pl.ds(s, 16)] = buf[r, pl.ds(s, 16)] + t
            return 0

        lax.fori_loop(0, NVREG_, vec_body, 0)
        pltpu.sync_copy(buf, out_hbm.at[pl.ds(off, CH_)])
        return 0

    lax.fori_loop(0, NCH_, chunk_body, 0)


def _kernel_sc(x, embeddings, modality_id):
    idx = jnp.full((16,), modality_id, dtype=jnp.int32)
    x2 = x.reshape(ROWS_, DIM_)
    mesh = plsc.VectorSubcoreMesh(core_axis_name="c", subcore_axis_name="s")
    out = pl.kernel(
        _sc_body,
        out_type=jax.ShapeDtypeStruct((ROWS_, DIM_), x.dtype),
        mesh=mesh,
        scratch_types=[
            pltpu.VMEM((16,), jnp.int32),
            pltpu.VMEM((16, DIM_), jnp.float32),
            pltpu.VMEM((CH_, DIM_), jnp.float32),
            pltpu.SemaphoreType.DMA,
        ],
    )(x2, embeddings, idx)
    return out.reshape(x.shape)


def _kernel(idx_ref, x_ref, emb_ref, o_ref):
    i = idx_ref[0]
    emb = emb_ref[:, :]
    row_ids = jax.lax.broadcasted_iota(jnp.int32, emb.shape, 0)
    tag = jnp.sum(jnp.where(row_ids == i, emb, 0.0), axis=0, keepdims=True)
    o_ref[:, :] = x_ref[:, :] + tag


def kernel(x, embeddings, modality_id):
    return _kernel_sc(x, embeddings, modality_id)


def _kernel_tc(x, embeddings, modality_id):
    idx = jnp.asarray(modality_id, dtype=jnp.int32).reshape((1,))
    x2 = x.reshape(ROWS_, DIM_)
    grid = ROWS_ // BLOCK_ROWS_
    out = pl.pallas_call(
        _kernel,
        grid_spec=pltpu.PrefetchScalarGridSpec(
            num_scalar_prefetch=1,
            grid=(grid,),
            in_specs=[
                pl.BlockSpec((BLOCK_ROWS_, DIM_), lambda g, s_ref: (g, 0)),
                pl.BlockSpec(embeddings.shape, lambda g, s_ref: (0, 0)),
            ],
            out_specs=pl.BlockSpec((BLOCK_ROWS_, DIM_), lambda g, s_ref: (g, 0)),
        ),
        out_shape=jax.ShapeDtypeStruct((ROWS_, DIM_), x.dtype),
    )(idx, x2, embeddings)
    return out.reshape(x.shape)


# speedup vs baseline: 1.4098x; 1.4098x over previous
import functools

import jax
import jax.numpy as jnp
from jax import lax
from jax.experimental import pallas as pl
from jax.experimental.pallas import tpu as pltpu
from jax.experimental.pallas import tpu_sc as plsc

DIM_ = 2048
ROWS_ = 4 * 4096
BLOCK_ROWS_ = 1024

NW_ = 32
RPW_ = ROWS_ // NW_
CH_ = 8
NCH_ = RPW_ // CH_
NBUF_ = 4
PD_ = 2
NVREG_ = DIM_ // 16


def _sc_body(x_hbm, emb_hbm, idx_hbm, out_hbm, idxv, tagblk,
             buf0, buf1, buf2, buf3, ls0, ls1, ls2, ls3, ss0, ss1, ss2, ss3):
    bufs = (buf0, buf1, buf2, buf3)
    lsems = (ls0, ls1, ls2, ls3)
    ssems = (ss0, ss1, ss2, ss3)
    wid = lax.axis_index("s") * 2 + lax.axis_index("c")
    base = wid * RPW_
    pltpu.sync_copy(idx_hbm, idxv)
    pltpu.async_copy(emb_hbm.at[idxv], tagblk, ls0).wait()

    def start_load(g, b):
        pltpu.async_copy(x_hbm.at[pl.ds(base + g * CH_, CH_)], bufs[b], lsems[b])

    def wait_load(b):
        pltpu.make_async_copy(x_hbm.at[pl.ds(base, CH_)], bufs[b], lsems[b]).wait()

    def start_store(g, b):
        pltpu.async_copy(bufs[b], out_hbm.at[pl.ds(base + g * CH_, CH_)], ssems[b])

    def wait_store(b):
        pltpu.make_async_copy(bufs[b], out_hbm.at[pl.ds(base, CH_)], ssems[b]).wait()

    def compute(b):
        buf = bufs[b]

        def vec_body(v, _):
            s = v * 16
            t = tagblk[0, pl.ds(s, 16)]
            for r in range(CH_):
                buf[r, pl.ds(s, 16)] = buf[r, pl.ds(s, 16)] + t
            return 0

        lax.fori_loop(0, NVREG_, vec_body, 0)

    for g in range(PD_):
        start_load(g, g % NBUF_)

    for b in range(NBUF_):
        g = b
        wait_load(b)
        compute(b)
        start_store(g, b)
        gp = g + PD_
        if gp < NCH_:
            if gp >= NBUF_:
                wait_store(gp % NBUF_)
            start_load(gp, gp % NBUF_)

    def group(i, _):
        for b in range(NBUF_):
            g = i * NBUF_ + b
            wait_load(b)
            compute(b)
            start_store(g, b)
            wait_store((b + PD_) % NBUF_)
            start_load(g + PD_, (b + PD_) % NBUF_)
        return 0

    lax.fori_loop(1, NCH_ // NBUF_ - 1, group, 0)

    for b in range(NBUF_):
        g = NCH_ - NBUF_ + b
        wait_load(b)
        compute(b)
        start_store(g, b)
        gp = g + PD_
        if gp < NCH_:
            wait_store(gp % NBUF_)
            start_load(gp, gp % NBUF_)
    for b in range(NBUF_):
        wait_store(b)


def _kernel_sc(x, embeddings, modality_id):
    idx = jnp.full((CH_,), modality_id, dtype=jnp.int32)
    x2 = x.reshape(ROWS_, DIM_)
    mesh = plsc.VectorSubcoreMesh(core_axis_name="c", subcore_axis_name="s")
    out = pl.kernel(
        _sc_body,
        out_type=jax.ShapeDtypeStruct((ROWS_, DIM_), x.dtype),
        mesh=mesh,
        scratch_types=[
            pltpu.VMEM((CH_,), jnp.int32),
            pltpu.VMEM((CH_, DIM_), jnp.float32),
        ]
        + [pltpu.VMEM((CH_, DIM_), jnp.float32)] * NBUF_
        + [pltpu.SemaphoreType.DMA] * (2 * NBUF_),
    )(x2, embeddings, idx)
    return out.reshape(x.shape)


def _kernel(idx_ref, x_ref, emb_ref, o_ref):
    i = idx_ref[0]
    emb = emb_ref[:, :]
    row_ids = jax.lax.broadcasted_iota(jnp.int32, emb.shape, 0)
    tag = jnp.sum(jnp.where(row_ids == i, emb, 0.0), axis=0, keepdims=True)
    o_ref[:, :] = x_ref[:, :] + tag


def kernel(x, embeddings, modality_id):
    return _kernel_sc(x, embeddings, modality_id)


def _kernel_tc(x, embeddings, modality_id):
    idx = jnp.asarray(modality_id, dtype=jnp.int32).reshape((1,))
    x2 = x.reshape(ROWS_, DIM_)
    grid = ROWS_ // BLOCK_ROWS_
    out = pl.pallas_call(
        _kernel,
        grid_spec=pltpu.PrefetchScalarGridSpec(
            num_scalar_prefetch=1,
            grid=(grid,),
            in_specs=[
                pl.BlockSpec((BLOCK_ROWS_, DIM_), lambda g, s_ref: (g, 0)),
                pl.BlockSpec(embeddings.shape, lambda g, s_ref: (0, 0)),
            ],
            out_specs=pl.BlockSpec((BLOCK_ROWS_, DIM_), lambda g, s_ref: (g, 0)),
        ),
        out_shape=jax.ShapeDtypeStruct((ROWS_, DIM_), x.dtype),
    )(idx, x2, embeddings)
    return out.reshape(x.shape)


# speedup vs baseline: 1.7704x; 1.2558x over previous
import functools

import jax
import jax.numpy as jnp
from jax import lax
from jax.experimental import pallas as pl
from jax.experimental.pallas import tpu as pltpu
from jax.experimental.pallas import tpu_sc as plsc

DIM_ = 2048
ROWS_ = 4 * 4096
BLOCK_ROWS_ = 1024

NW_ = 32
RPW_ = ROWS_ // NW_
CH_ = 8
NCH_ = RPW_ // CH_
NBUF_ = 4
PD_ = 2
NVREG_ = DIM_ // 16


def _sc_body(x_hbm, emb_hbm, idx_hbm, out_hbm, idxv, tagblk,
             buf0, buf1, buf2, buf3, ls0, ls1, ls2, ls3, ss0, ss1, ss2, ss3):
    bufs = (buf0, buf1, buf2, buf3)
    lsems = (ls0, ls1, ls2, ls3)
    ssems = (ss0, ss1, ss2, ss3)
    wid = lax.axis_index("s") * 2 + lax.axis_index("c")
    base = wid * RPW_
    pltpu.sync_copy(idx_hbm, idxv)
    pltpu.async_copy(emb_hbm.at[idxv], tagblk, ls0).wait()

    def start_load(g, b):
        pltpu.async_copy(x_hbm.at[pl.ds(base + g * CH_, CH_)], bufs[b], lsems[b])

    def wait_load(b):
        pltpu.make_async_copy(x_hbm.at[pl.ds(base, CH_)], bufs[b], lsems[b]).wait()

    def start_store(g, b):
        pltpu.async_copy(bufs[b], out_hbm.at[pl.ds(base + g * CH_, CH_)], ssems[b])

    def wait_store(b):
        pltpu.make_async_copy(bufs[b], out_hbm.at[pl.ds(base, CH_)], ssems[b]).wait()

    def compute(b):
        buf = bufs[b]

        def vec_body(v, _):
            s0 = v * (8 * 16)
            for u in range(8):
                s = s0 + u * 16
                t = tagblk[0, pl.ds(s, 16)]
                for r in range(CH_):
                    buf[r, pl.ds(s, 16)] = buf[r, pl.ds(s, 16)] + t
            return 0

        lax.fori_loop(0, NVREG_ // 8, vec_body, 0)

    for g in range(PD_):
        start_load(g, g % NBUF_)

    for b in range(NBUF_):
        g = b
        wait_load(b)
        compute(b)
        start_store(g, b)
        gp = g + PD_
        if gp < NCH_:
            if gp >= NBUF_:
                wait_store(gp % NBUF_)
            start_load(gp, gp % NBUF_)

    def group(i, _):
        for b in range(NBUF_):
            g = i * NBUF_ + b
            wait_load(b)
            compute(b)
            start_store(g, b)
            wait_store((b + PD_) % NBUF_)
            start_load(g + PD_, (b + PD_) % NBUF_)
        return 0

    lax.fori_loop(1, NCH_ // NBUF_ - 1, group, 0)

    for b in range(NBUF_):
        g = NCH_ - NBUF_ + b
        wait_load(b)
        compute(b)
        start_store(g, b)
        gp = g + PD_
        if gp < NCH_:
            wait_store(gp % NBUF_)
            start_load(gp, gp % NBUF_)
    for b in range(NBUF_):
        wait_store(b)


def _kernel_sc(x, embeddings, modality_id):
    idx = jnp.full((CH_,), modality_id, dtype=jnp.int32)
    x2 = x.reshape(ROWS_, DIM_)
    mesh = plsc.VectorSubcoreMesh(core_axis_name="c", subcore_axis_name="s")
    out = pl.kernel(
        _sc_body,
        out_type=jax.ShapeDtypeStruct((ROWS_, DIM_), x.dtype),
        mesh=mesh,
        scratch_types=[
            pltpu.VMEM((CH_,), jnp.int32),
            pltpu.VMEM((CH_, DIM_), jnp.float32),
        ]
        + [pltpu.VMEM((CH_, DIM_), jnp.float32)] * NBUF_
        + [pltpu.SemaphoreType.DMA] * (2 * NBUF_),
    )(x2, embeddings, idx)
    return out.reshape(x.shape)


def _kernel(idx_ref, x_ref, emb_ref, o_ref):
    i = idx_ref[0]
    emb = emb_ref[:, :]
    row_ids = jax.lax.broadcasted_iota(jnp.int32, emb.shape, 0)
    tag = jnp.sum(jnp.where(row_ids == i, emb, 0.0), axis=0, keepdims=True)
    o_ref[:, :] = x_ref[:, :] + tag


def kernel(x, embeddings, modality_id):
    return _kernel_sc(x, embeddings, modality_id)


def _kernel_tc(x, embeddings, modality_id):
    idx = jnp.asarray(modality_id, dtype=jnp.int32).reshape((1,))
    x2 = x.reshape(ROWS_, DIM_)
    grid = ROWS_ // BLOCK_ROWS_
    out = pl.pallas_call(
        _kernel,
        grid_spec=pltpu.PrefetchScalarGridSpec(
            num_scalar_prefetch=1,
            grid=(grid,),
            in_specs=[
                pl.BlockSpec((BLOCK_ROWS_, DIM_), lambda g, s_ref: (g, 0)),
                pl.BlockSpec(embeddings.shape, lambda g, s_ref: (0, 0)),
            ],
            out_specs=pl.BlockSpec((BLOCK_ROWS_, DIM_), lambda g, s_ref: (g, 0)),
        ),
        out_shape=jax.ShapeDtypeStruct((ROWS_, DIM_), x.dtype),
    )(idx, x2, embeddings)
    return out.reshape(x.shape)
